# Initial kernel scaffold; baseline (speedup 1.0000x reference)
import jax
import jax.numpy as jnp
from jax.experimental import pallas as pl


def kernel(x, edge_index, edge_attr, mu1, sigma1, g1, root1, bias1, mu2, sigma2, g2, root2, bias2):
    raise NotImplementedError("write your pallas kernel here")


# speedup vs baseline: 2.4546x; 2.4546x over previous
import functools

import jax
import jax.numpy as jnp
from jax import lax
from jax.experimental import pallas as pl
from jax.experimental.pallas import tpu as pltpu
from jax.experimental.pallas import tpu_sc as plsc

_N = 10000
_E = 320000
_F = 128
_H = 64
_C = 10
_K = 3
_D = 2

_NC = 2
_NS = 16
_NW = _NC * _NS
_CH = 128
_CPW = 79
_EW = _CH * _CPW
_EP = _EW * _NW
_WC = 16
_RPT = _N // _NS
_RB = 125


def _bcast_lane(v, k):
  idx = jnp.full((16, 1), k, jnp.int32)
  dn = lax.GatherDimensionNumbers(
      offset_dims=(), collapsed_slice_dims=(0,), start_index_map=(0,))
  return lax.gather(v, idx, dn, (1,),
                    mode=lax.GatherScatterMode.PROMISE_IN_BOUNDS)


def _sc_edge_aggregate(feats, src3, dst3, w3, fw, woff, aggw, with_cnt):
  mesh = plsc.VectorSubcoreMesh(core_axis_name="c", subcore_axis_name="s")
  out_w = fw // _K
  nob = out_w // 16
  nmsg = aggw // 16

  def body(f_hbm, src_hbm, dst_hbm, w_hbm, out_hbm,
           src_v, dst_v, w_v, rows_v, msg_v, agg_sh, sem):
    cid = lax.axis_index("c")
    sid = lax.axis_index("s")
    wid = cid * _NS + sid
    onehot0 = jnp.where(lax.iota(jnp.int32, 16) == 0,
                        jnp.float32(1.0), jnp.float32(0.0))

    def zrow(i, carry):
      for j in range(nmsg):
        msg_v[i, pl.ds(j * 16, 16)] = jnp.zeros((16,), jnp.float32)
      return carry
    lax.fori_loop(0, _RB, zrow, 0)
    base = sid * _RPT
    for r in range(_RPT // _RB):
      pltpu.sync_copy(msg_v.at[pl.ds(0, _RB)],
                      agg_sh.at[pl.ds(base + r * _RB, _RB)])
    plsc.subcore_barrier()

    pltpu.sync_copy(src_hbm.at[wid], src_v)
    pltpu.sync_copy(dst_hbm.at[wid], dst_v)

    def chunk(c, carry):
      pltpu.async_copy(f_hbm.at[src_v.at[c]], rows_v, sem).wait()
      pltpu.sync_copy(w_hbm.at[wid, c], w_v)

      def edge(e, carry2):
        wv = w_v[e, :]
        wk = [_bcast_lane(wv, woff + t) for t in range(_K)]
        for j in range(nob):
          acc = wk[0] * rows_v[e, pl.ds(j * 16, 16)]
          for t in range(1, _K):
            acc = acc + wk[t] * rows_v[e, pl.ds(t * out_w + j * 16, 16)]
          msg_v[e, pl.ds(j * 16, 16)] = acc
        if with_cnt:
          wc = _bcast_lane(wv, 3)
          msg_v[e, pl.ds(out_w, 16)] = wc * onehot0
        return carry2

      lax.fori_loop(0, _CH, edge, 0)
      pltpu.sync_copy(msg_v, agg_sh.at[dst_v.at[c]], add=True)
      return carry

    lax.fori_loop(0, _CPW, chunk, 0)
    plsc.subcore_barrier()
    for r in range(_RPT // _RB):
      off = base + r * _RB
      pltpu.sync_copy(agg_sh.at[pl.ds(off, _RB)],
                      out_hbm.at[cid, pl.ds(off, _RB)])

  f = pl.kernel(
      body,
      out_type=jax.ShapeDtypeStruct((_NC, _N, aggw), jnp.float32),
      mesh=mesh,
      compiler_params=pltpu.CompilerParams(use_tc_tiling_on_sc=False),
      scratch_types=[
          pltpu.VMEM((_CPW, _CH), jnp.int32),
          pltpu.VMEM((_CPW, _CH), jnp.int32),
          pltpu.VMEM((_CH, _WC), jnp.float32),
          pltpu.VMEM((_CH, fw), jnp.float32),
          pltpu.VMEM((_CH, aggw), jnp.float32),
          pltpu.VMEM_SHARED((_N, aggw), jnp.float32),
          pltpu.SemaphoreType.DMA,
      ],
  )
  return f(feats, src3, dst3, w3)


def _tc_mm1(x, g1, root1):
  blk = 1000

  def body(x_ref, g_ref, r_ref, xg_ref, xr_ref):
    xb = x_ref[...]
    xg_ref[...] = jnp.dot(xb, g_ref[...], preferred_element_type=jnp.float32)
    xr_ref[...] = jnp.dot(xb, r_ref[...], preferred_element_type=jnp.float32)

  return pl.pallas_call(
      body,
      grid=(_N // blk,),
      in_specs=[
          pl.BlockSpec((blk, _F), lambda i: (i, 0)),
          pl.BlockSpec((_F, _H * _K), lambda i: (0, 0)),
          pl.BlockSpec((_F, _H), lambda i: (0, 0)),
      ],
      out_specs=[
          pl.BlockSpec((blk, _H * _K), lambda i: (i, 0)),
          pl.BlockSpec((blk, _H), lambda i: (i, 0)),
      ],
      out_shape=[
          jax.ShapeDtypeStruct((_N, _H * _K), jnp.float32),
          jax.ShapeDtypeStruct((_N, _H), jnp.float32),
      ],
  )(x, g1, root1)


def _tc_weights(eap, q):
  blk = 2048

  def body(ea_ref, q_ref, w_ref):
    ea = ea_ref[...]
    ea0 = ea[:, 0:1]
    ea1 = ea[:, 1:2]
    qq = q_ref[...]
    logw = ((ea0 * ea0) * qq[0:1] + (ea1 * ea1) * qq[1:2]
            + ea0 * qq[2:3] + ea1 * qq[3:4] + qq[4:5])
    w = jnp.exp(logw)
    row = (pl.program_id(0) * blk
           + lax.broadcasted_iota(jnp.int32, (blk, _WC), 0))
    w_ref[...] = jnp.where(row < _E, w, 0.0)

  return pl.pallas_call(
      body,
      grid=(_EP // blk,),
      in_specs=[
          pl.BlockSpec((blk, _D), lambda i: (i, 0)),
          pl.BlockSpec((8, _WC), lambda i: (0, 0)),
      ],
      out_specs=pl.BlockSpec((blk, _WC), lambda i: (i, 0)),
      out_shape=jax.ShapeDtypeStruct((_EP, _WC), jnp.float32),
  )(eap, q)


def _tc_combine1(p, xr, b1, g2p, r2p):
  blk = 1000

  def body(p_ref, xr_ref, b_ref, g_ref, r_ref, hg_ref, hr_ref, cnt_ref):
    s = p_ref[0] + p_ref[1]
    cnt = jnp.maximum(s[:, _H:_H + 1], 1.0)
    pre = s[:, :_H] / cnt + xr_ref[...] + b_ref[...]
    h = jnp.where(pre > 0, pre, jnp.exp(jnp.minimum(pre, 0.0)) - 1.0)
    hg_ref[...] = jnp.dot(h, g_ref[...], preferred_element_type=jnp.float32)
    hr_ref[...] = jnp.dot(h, r_ref[...], preferred_element_type=jnp.float32)
    cnt_ref[...] = jnp.broadcast_to(cnt, (blk, 16))

  return pl.pallas_call(
      body,
      grid=(_N // blk,),
      in_specs=[
          pl.BlockSpec((_NC, blk, _H + 16), lambda i: (0, i, 0)),
          pl.BlockSpec((blk, _H), lambda i: (i, 0)),
          pl.BlockSpec((1, _H), lambda i: (0, 0)),
          pl.BlockSpec((_H, _K * 16), lambda i: (0, 0)),
          pl.BlockSpec((_H, 16), lambda i: (0, 0)),
      ],
      out_specs=[
          pl.BlockSpec((blk, _K * 16), lambda i: (i, 0)),
          pl.BlockSpec((blk, 16), lambda i: (i, 0)),
          pl.BlockSpec((blk, 16), lambda i: (i, 0)),
      ],
      out_shape=[
          jax.ShapeDtypeStruct((_N, _K * 16), jnp.float32),
          jax.ShapeDtypeStruct((_N, 16), jnp.float32),
          jax.ShapeDtypeStruct((_N, 16), jnp.float32),
      ],
  )(p, xr, b1, g2p, r2p)


def _tc_combine2(q, hr, cnt16, b2):
  blk = 1000

  def body(q_ref, hr_ref, c_ref, b_ref, o_ref):
    s = q_ref[0] + q_ref[1]
    o = s / c_ref[...] + hr_ref[...] + b_ref[...]
    valid = lax.broadcasted_iota(jnp.int32, (blk, 16), 1) < _C
    om = jnp.where(valid, o, -1e30)
    m = jnp.max(om, axis=1, keepdims=True)
    z = jnp.where(valid, jnp.exp(o - m), 0.0)
    lse = jnp.log(jnp.sum(z, axis=1, keepdims=True))
    o_ref[...] = o - m - lse

  return pl.pallas_call(
      body,
      grid=(_N // blk,),
      in_specs=[
          pl.BlockSpec((_NC, blk, 16), lambda i: (0, i, 0)),
          pl.BlockSpec((blk, 16), lambda i: (i, 0)),
          pl.BlockSpec((blk, 16), lambda i: (i, 0)),
          pl.BlockSpec((1, 16), lambda i: (0, 0)),
      ],
      out_specs=pl.BlockSpec((blk, 16), lambda i: (i, 0)),
      out_shape=jax.ShapeDtypeStruct((_N, 16), jnp.float32),
  )(q, hr, cnt16, b2)


def _mixture_coeffs(mu, sigma):
  s2 = sigma * sigma + 1e-14
  a2 = -0.5 / s2
  a1 = mu / s2
  a0 = -0.5 * jnp.sum(mu * mu / s2, axis=1)
  return jnp.stack([a2[:, 0], a2[:, 1], a1[:, 0], a1[:, 1], a0], axis=0)


def kernel(x, edge_index, edge_attr, mu1, sigma1, g1, root1, bias1,
           mu2, sigma2, g2, root2, bias2):
  q1 = _mixture_coeffs(mu1, sigma1)
  q2 = _mixture_coeffs(mu2, sigma2)
  tail = jnp.zeros((5, _WC - 2 * _K - 1), jnp.float32).at[4].set(-1e30)
  qtop = jnp.concatenate(
      [q1, jnp.zeros((5, 1), jnp.float32), q2, tail], axis=1)
  q = jnp.concatenate([qtop, jnp.zeros((3, _WC), jnp.float32)], axis=0)

  pad = _EP - _E
  src3 = jnp.concatenate(
      [edge_index[0], jnp.zeros((pad,), jnp.int32)]).reshape(_NW, _CPW, _CH)
  dst3 = jnp.concatenate(
      [edge_index[1], jnp.zeros((pad,), jnp.int32)]).reshape(_NW, _CPW, _CH)
  eap = jnp.concatenate([edge_attr, jnp.zeros((pad, _D), jnp.float32)])

  g2p = jnp.zeros((_H, _K, 16), jnp.float32).at[:, :, :_C].set(
      g2.reshape(_H, _K, _C)).reshape(_H, _K * 16)
  r2p = jnp.zeros((_H, 16), jnp.float32).at[:, :_C].set(root2)
  b2p = jnp.zeros((1, 16), jnp.float32).at[0, :_C].set(bias2)
  b1p = bias1.reshape(1, _H)

  w = _tc_weights(eap, q).reshape(_NW, _CPW, _CH, _WC)
  xg, xr = _tc_mm1(x, g1, root1)
  p1 = _sc_edge_aggregate(xg, src3, dst3, w, _H * _K, 0, _H + 16, True)
  hg, hr, cnt16 = _tc_combine1(p1, xr, b1p, g2p, r2p)
  p2 = _sc_edge_aggregate(hg, src3, dst3, w, _K * 16, 4, 16, False)
  out16 = _tc_combine2(p2, hr, cnt16, b2p)
  return out16[:, :_C]
